# baseline (speedup 1.0000x reference)
import jax
import jax.numpy as jnp
from jax.experimental import pallas as pl

GRID = 4
WIN = 3
STRIDE = GRID - WIN + 1
NC = 4


def _band_sum_kernel(x_ref, o_ref):
    band = x_ref.shape[2] // GRID
    first = pl.program_id(1) == 0
    for g in range(GRID):
        s = jnp.sum(x_ref[0, :, g * band:(g + 1) * band, :], axis=(0, 1))

        @pl.when(first)
        def _init(g=g, s=s):
            o_ref[0, g, :] = s

        @pl.when(jnp.logical_not(first))
        def _acc(g=g, s=s):
            o_ref[0, g, :] += s


def _select_kernel(r_ref, o_ref):
    x = r_ref[...]
    B, G, W = x.shape
    gw = W // GRID
    lane = jax.lax.broadcasted_iota(jnp.int32, (B, W), 1)
    ws = []
    for i in range(STRIDE):
        rows = jnp.sum(x[:, i:i + WIN, :], axis=1)
        for j in range(STRIDE):
            m = (lane >= j * gw) & (lane < (j + WIN) * gw)
            ws.append(jnp.sum(jnp.where(m, rows, 0.0), axis=1))
    best_val = ws[0]
    best_idx = jnp.zeros((B,), jnp.int32)
    for k in range(1, STRIDE * STRIDE):
        better = ws[k] > best_val
        best_val = jnp.where(better, ws[k], best_val)
        best_idx = jnp.where(better, k, best_idx)
    coords = jnp.concatenate(
        [(best_idx // STRIDE)[:, None], (best_idx % STRIDE)[:, None]], axis=1)
    o_ref[...] = coords.astype(jnp.int32)


def kernel(sampling_map):
    B, C, H, W = sampling_map.shape
    cchunk = C // NC

    band_sums = pl.pallas_call(
        _band_sum_kernel,
        grid=(B, NC),
        in_specs=[pl.BlockSpec((1, cchunk, H, W), lambda b, c: (b, c, 0, 0))],
        out_specs=pl.BlockSpec((1, GRID, W), lambda b, c: (b, 0, 0)),
        out_shape=jax.ShapeDtypeStruct((B, GRID, W), jnp.float32),
    )(sampling_map)

    coords = pl.pallas_call(
        _select_kernel,
        in_specs=[pl.BlockSpec((B, GRID, W), lambda: (0, 0, 0))],
        out_specs=pl.BlockSpec((B, 2), lambda: (0, 0)),
        out_shape=jax.ShapeDtypeStruct((B, 2), jnp.int32),
    )(band_sums)

    return coords.reshape(B, 1, 2)


# speedup vs baseline: 1.0326x; 1.0326x over previous
import jax
import jax.numpy as jnp
from jax.experimental import pallas as pl

GRID = 4
WIN = 3
STRIDE = GRID - WIN + 1


def _band_sum_kernel(x_ref, o_ref):
    o_ref[0, 0, 0, :] = jnp.sum(x_ref[...], axis=(0, 1, 2))


def _select_kernel(r_ref, o_ref):
    x = r_ref[...]
    B, G, W = x.shape
    gw = W // GRID
    lane = jax.lax.broadcasted_iota(jnp.int32, (B, W), 1)
    ws = []
    for i in range(STRIDE):
        rows = jnp.sum(x[:, i:i + WIN, :], axis=1)
        for j in range(STRIDE):
            m = (lane >= j * gw) & (lane < (j + WIN) * gw)
            ws.append(jnp.sum(jnp.where(m, rows, 0.0), axis=1))
    best_val = ws[0]
    best_idx = jnp.zeros((B,), jnp.int32)
    for k in range(1, STRIDE * STRIDE):
        better = ws[k] > best_val
        best_val = jnp.where(better, ws[k], best_val)
        best_idx = jnp.where(better, k, best_idx)
    coords = jnp.concatenate(
        [(best_idx // STRIDE)[:, None], (best_idx % STRIDE)[:, None]], axis=1)
    o_ref[...] = coords.astype(jnp.int32)


def kernel(sampling_map):
    B, C, H, W = sampling_map.shape
    band = H // GRID

    band_sums = pl.pallas_call(
        _band_sum_kernel,
        grid=(B, GRID),
        in_specs=[pl.BlockSpec((1, C, band, W), lambda b, g: (b, 0, g, 0))],
        out_specs=pl.BlockSpec((1, 1, 1, W), lambda b, g: (b, g, 0, 0)),
        out_shape=jax.ShapeDtypeStruct((B, GRID, 1, W), jnp.float32),
    )(sampling_map)
    band_sums = band_sums.reshape(B, GRID, W)

    coords = pl.pallas_call(
        _select_kernel,
        in_specs=[pl.BlockSpec((B, GRID, W), lambda: (0, 0, 0))],
        out_specs=pl.BlockSpec((B, 2), lambda: (0, 0)),
        out_shape=jax.ShapeDtypeStruct((B, 2), jnp.int32),
    )(band_sums)

    return coords.reshape(B, 1, 2)


# speedup vs baseline: 1.0343x; 1.0016x over previous
import jax
import jax.numpy as jnp
from jax.experimental import pallas as pl

GRID = 4
WIN = 3
STRIDE = GRID - WIN + 1


def _band_sum_kernel(x_ref, y_ref, o_ref):
    o_ref[0, 0, 0, :] = (jnp.sum(x_ref[...], axis=(0, 1, 2))
                         + jnp.sum(y_ref[...], axis=(0, 1, 2)))


def _select_kernel(r_ref, o_ref):
    x = r_ref[...]
    B, G, W = x.shape
    gw = W // GRID
    lane = jax.lax.broadcasted_iota(jnp.int32, (B, W), 1)
    ws = []
    for i in range(STRIDE):
        rows = jnp.sum(x[:, i:i + WIN, :], axis=1)
        for j in range(STRIDE):
            m = (lane >= j * gw) & (lane < (j + WIN) * gw)
            ws.append(jnp.sum(jnp.where(m, rows, 0.0), axis=1))
    best_val = ws[0]
    best_idx = jnp.zeros((B,), jnp.int32)
    for k in range(1, STRIDE * STRIDE):
        better = ws[k] > best_val
        best_val = jnp.where(better, ws[k], best_val)
        best_idx = jnp.where(better, k, best_idx)
    coords = jnp.concatenate(
        [(best_idx // STRIDE)[:, None], (best_idx % STRIDE)[:, None]], axis=1)
    o_ref[...] = coords.astype(jnp.int32)


def kernel(sampling_map):
    B, C, H, W = sampling_map.shape
    band = H // GRID

    band_sums = pl.pallas_call(
        _band_sum_kernel,
        grid=(B, GRID),
        in_specs=[
            pl.BlockSpec((1, C // 2, band, W), lambda b, g: (b, 0, g, 0)),
            pl.BlockSpec((1, C // 2, band, W), lambda b, g: (b, 1, g, 0)),
        ],
        out_specs=pl.BlockSpec((1, 1, 1, W), lambda b, g: (b, g, 0, 0)),
        out_shape=jax.ShapeDtypeStruct((B, GRID, 1, W), jnp.float32),
    )(sampling_map, sampling_map)
    band_sums = band_sums.reshape(B, GRID, W)

    coords = pl.pallas_call(
        _select_kernel,
        in_specs=[pl.BlockSpec((B, GRID, W), lambda: (0, 0, 0))],
        out_specs=pl.BlockSpec((B, 2), lambda: (0, 0)),
        out_shape=jax.ShapeDtypeStruct((B, 2), jnp.int32),
    )(band_sums)

    return coords.reshape(B, 1, 2)


# speedup vs baseline: 1.0361x; 1.0017x over previous
import jax
import jax.numpy as jnp
from jax.experimental import pallas as pl

GRID = 4
WIN = 3
STRIDE = GRID - WIN + 1


def _band_sum_kernel(x_ref, y_ref, z_ref, o_ref):
    o_ref[0, 0, 0, :] = (jnp.sum(x_ref[...], axis=(0, 1, 2))
                         + jnp.sum(y_ref[...], axis=(0, 1, 2))
                         + jnp.sum(z_ref[...], axis=(0, 1, 2)))


def _select_kernel(r_ref, o_ref):
    x = r_ref[...]
    B, G, W = x.shape
    gw = W // GRID
    lane = jax.lax.broadcasted_iota(jnp.int32, (B, W), 1)
    ws = []
    for i in range(STRIDE):
        rows = jnp.sum(x[:, i:i + WIN, :], axis=1)
        for j in range(STRIDE):
            m = (lane >= j * gw) & (lane < (j + WIN) * gw)
            ws.append(jnp.sum(jnp.where(m, rows, 0.0), axis=1))
    best_val = ws[0]
    best_idx = jnp.zeros((B,), jnp.int32)
    for k in range(1, STRIDE * STRIDE):
        better = ws[k] > best_val
        best_val = jnp.where(better, ws[k], best_val)
        best_idx = jnp.where(better, k, best_idx)
    coords = jnp.concatenate(
        [(best_idx // STRIDE)[:, None], (best_idx % STRIDE)[:, None]], axis=1)
    o_ref[...] = coords.astype(jnp.int32)


def kernel(sampling_map):
    B, C, H, W = sampling_map.shape
    band = H // GRID

    band_sums = pl.pallas_call(
        _band_sum_kernel,
        grid=(B, GRID),
        in_specs=[
            pl.BlockSpec((1, C // 3, band, W), lambda b, g: (b, 0, g, 0)),
            pl.BlockSpec((1, C // 3, band, W), lambda b, g: (b, 1, g, 0)),
            pl.BlockSpec((1, C // 3, band, W), lambda b, g: (b, 2, g, 0)),
        ],
        out_specs=pl.BlockSpec((1, 1, 1, W), lambda b, g: (b, g, 0, 0)),
        out_shape=jax.ShapeDtypeStruct((B, GRID, 1, W), jnp.float32),
    )(sampling_map, sampling_map, sampling_map)
    band_sums = band_sums.reshape(B, GRID, W)

    coords = pl.pallas_call(
        _select_kernel,
        in_specs=[pl.BlockSpec((B, GRID, W), lambda: (0, 0, 0))],
        out_specs=pl.BlockSpec((B, 2), lambda: (0, 0)),
        out_shape=jax.ShapeDtypeStruct((B, 2), jnp.int32),
    )(band_sums)

    return coords.reshape(B, 1, 2)


# speedup vs baseline: 1.0371x; 1.0010x over previous
import jax
import jax.numpy as jnp
from jax.experimental import pallas as pl

GRID = 4
WIN = 3
STRIDE = GRID - WIN + 1
N_STREAMS = 4


def _band_sum_kernel(*refs):
    o_ref = refs[-1]
    o_ref[0, 0, 0, :] = sum(jnp.sum(r[...], axis=(0, 1, 2)) for r in refs[:-1])


def _select_kernel(r_ref, o_ref):
    x = r_ref[...]
    B, G, W = x.shape
    gw = W // GRID
    lane = jax.lax.broadcasted_iota(jnp.int32, (B, W), 1)
    ws = []
    for i in range(STRIDE):
        rows = jnp.sum(x[:, i:i + WIN, :], axis=1)
        for j in range(STRIDE):
            m = (lane >= j * gw) & (lane < (j + WIN) * gw)
            ws.append(jnp.sum(jnp.where(m, rows, 0.0), axis=1))
    best_val = ws[0]
    best_idx = jnp.zeros((B,), jnp.int32)
    for k in range(1, STRIDE * STRIDE):
        better = ws[k] > best_val
        best_val = jnp.where(better, ws[k], best_val)
        best_idx = jnp.where(better, k, best_idx)
    coords = jnp.concatenate(
        [(best_idx // STRIDE)[:, None], (best_idx % STRIDE)[:, None]], axis=1)
    o_ref[...] = coords.astype(jnp.int32)


def kernel(sampling_map):
    B, C, H, W = sampling_map.shape
    band = H // GRID

    band_sums = pl.pallas_call(
        _band_sum_kernel,
        grid=(B, GRID),
        in_specs=[
            pl.BlockSpec((1, C // N_STREAMS, band, W),
                         (lambda b, g, k=k: (b, k, g, 0)))
            for k in range(N_STREAMS)
        ],
        out_specs=pl.BlockSpec((1, 1, 1, W), lambda b, g: (b, g, 0, 0)),
        out_shape=jax.ShapeDtypeStruct((B, GRID, 1, W), jnp.float32),
    )(*([sampling_map] * N_STREAMS))
    band_sums = band_sums.reshape(B, GRID, W)

    coords = pl.pallas_call(
        _select_kernel,
        in_specs=[pl.BlockSpec((B, GRID, W), lambda: (0, 0, 0))],
        out_specs=pl.BlockSpec((B, 2), lambda: (0, 0)),
        out_shape=jax.ShapeDtypeStruct((B, 2), jnp.int32),
    )(band_sums)

    return coords.reshape(B, 1, 2)
